# Initial kernel scaffold; baseline (speedup 1.0000x reference)
import jax
import jax.numpy as jnp
from jax.experimental import pallas as pl


def kernel(x, edges, masks, W_ih_f, W_hh_f, b_ih_f, b_hh_f, W_ih_b, W_hh_b, b_ih_b, b_hh_b, gcn1_W, gcn1_b, gcn2_W, gcn2_b, cls_W, cls_b):
    raise NotImplementedError("write your pallas kernel here")


# speedup vs baseline: 1.5698x; 1.5698x over previous
import functools

import jax
import jax.numpy as jnp
from jax.experimental import pallas as pl

B = 2
N = 1000
T = 16
F_IN = 64
H = 64
C = 12
NP = 1024
R = 400


def _gru_kernel(xc_ref, wih_ref, whh_ref, bih_ref, bhh_ref, out_ref):
    xc = xc_ref[...]
    gx = jnp.dot(xc.reshape(T * R, 2 * F_IN), wih_ref[...],
                 preferred_element_type=jnp.float32) + bih_ref[...]
    gx = gx.reshape(T, R, 6 * H)
    whh = whh_ref[...]
    bhh = bhh_ref[...]
    h = jnp.zeros((R, 2 * H), jnp.float32)
    acc = jnp.zeros((R, 2 * H), jnp.float32)
    for t in range(T):
        gh = jnp.dot(h, whh, preferred_element_type=jnp.float32) + bhh
        gxt = gx[t]
        r = jax.nn.sigmoid(gxt[:, 0:128] + gh[:, 0:128])
        z = jax.nn.sigmoid(gxt[:, 128:256] + gh[:, 128:256])
        n = jnp.tanh(gxt[:, 256:384] + r * gh[:, 256:384])
        h = (1.0 - z) * n + z * h
        acc = acc + h
    out_ref[...] = acc * (1.0 / T)


def _gcn_kernel(at_ref, tmt_ref, w1t_ref, b1_ref, w2t_ref, b2_ref,
                cw_ref, cb_ref, mask_ref, out_ref):
    at = at_ref[...]
    w1t = w1t_ref[...]
    w2t = w2t_ref[...]
    b1 = b1_ref[...]
    b2 = b2_ref[...]
    cw = cw_ref[...]
    cb = cb_ref[0, 0]
    maskt = mask_ref[...]
    for b in range(B):
        tmt = tmt_ref[b]
        u1 = jnp.dot(tmt, at, preferred_element_type=jnp.float32)
        h1 = jnp.maximum(jnp.dot(w1t, u1, preferred_element_type=jnp.float32) + b1, 0.0)
        u2 = jnp.dot(h1, at, preferred_element_type=jnp.float32)
        h2 = jnp.maximum(jnp.dot(w2t, u2, preferred_element_type=jnp.float32) + b2, 0.0)
        logits = jnp.dot(cw, h2, preferred_element_type=jnp.float32) + cb
        masked = jnp.where(maskt == 0, -1e9, logits)
        m = jnp.max(masked, axis=1, keepdims=True)
        sh = masked - m
        lse = jnp.log(jnp.sum(jnp.exp(sh), axis=1, keepdims=True))
        out_ref[b] = sh - lse


def _blkdiag(a, b):
    z = jnp.zeros_like(a)
    return jnp.concatenate(
        [jnp.concatenate([a, z], axis=1), jnp.concatenate([z, b], axis=1)], axis=0)


@jax.jit
def kernel(x, edges, masks, W_ih_f, W_hh_f, b_ih_f, b_hh_f,
           W_ih_b, W_hh_b, b_ih_b, b_hh_b,
           gcn1_W, gcn1_b, gcn2_W, gcn2_b, cls_W, cls_b):
    wih_f, wih_b = W_ih_f.T, W_ih_b.T
    whh_f, whh_b = W_hh_f.T, W_hh_b.T
    wih = jnp.concatenate(
        [_blkdiag(wih_f[:, i * H:(i + 1) * H], wih_b[:, i * H:(i + 1) * H])
         for i in range(3)], axis=1)
    whh = jnp.concatenate(
        [_blkdiag(whh_f[:, i * H:(i + 1) * H], whh_b[:, i * H:(i + 1) * H])
         for i in range(3)], axis=1)
    bih = jnp.concatenate(
        [jnp.concatenate([b_ih_f[i * H:(i + 1) * H], b_ih_b[i * H:(i + 1) * H]])
         for i in range(3)]).reshape(1, 6 * H)
    bhh = jnp.concatenate(
        [jnp.concatenate([b_hh_f[i * H:(i + 1) * H], b_hh_b[i * H:(i + 1) * H]])
         for i in range(3)]).reshape(1, 6 * H)

    xs = x.reshape(B * N, T, F_IN)
    xc = jnp.concatenate([xs, xs[:, ::-1, :]], axis=-1)
    xct = xc.transpose(1, 0, 2)

    grid = (B * N) // R
    temporal = pl.pallas_call(
        _gru_kernel,
        grid=(grid,),
        in_specs=[
            pl.BlockSpec((T, R, 2 * F_IN), lambda i: (0, i, 0)),
            pl.BlockSpec((2 * F_IN, 6 * H), lambda i: (0, 0)),
            pl.BlockSpec((2 * H, 6 * H), lambda i: (0, 0)),
            pl.BlockSpec((1, 6 * H), lambda i: (0, 0)),
            pl.BlockSpec((1, 6 * H), lambda i: (0, 0)),
        ],
        out_specs=pl.BlockSpec((R, 2 * H), lambda i: (i, 0)),
        out_shape=jax.ShapeDtypeStruct((B * N, 2 * H), jnp.float32),
    )(xct, wih, whh, bih, bhh)

    at = jnp.pad(edges.T, ((0, NP - N), (0, NP - N)))
    tmt = jnp.pad(temporal.reshape(B, N, 2 * H), ((0, 0), (0, NP - N), (0, 0)))
    tmt = tmt.transpose(0, 2, 1)
    maskt = jnp.pad(masks.T.astype(jnp.int32), ((0, 0), (0, NP - N)))

    preds_pad = pl.pallas_call(
        _gcn_kernel,
        out_shape=jax.ShapeDtypeStruct((B, C, NP), jnp.float32),
    )(at, tmt, gcn1_W.T, gcn1_b.reshape(2 * H, 1), gcn2_W.T,
      gcn2_b.reshape(2 * H, 1), cls_W.T, cls_b.reshape(1, 1), maskt)

    return preds_pad[:, :, :N]


# speedup vs baseline: 2.4182x; 1.5405x over previous
import jax
import jax.numpy as jnp
from jax.experimental import pallas as pl

B = 2
N = 1000
T = 16
F_IN = 64
H = 64
C = 12
R = 400


def _gru_kernel(xct_ref, wih_ref, whh_ref, bih_ref, bhh_ref, out_ref):
    xbt = xct_ref[...]
    xrev = jnp.concatenate([xbt[T - 1 - t:T - t] for t in range(T)], axis=0)
    xc = jnp.concatenate([xbt, xrev], axis=-1)
    gx = jnp.dot(xc.reshape(T * R, 2 * F_IN), wih_ref[...],
                 preferred_element_type=jnp.float32) + bih_ref[...]
    gx = gx.reshape(T, R, 6 * H)
    whh = whh_ref[...]
    bhh = bhh_ref[...]
    h = jnp.zeros((R, 2 * H), jnp.float32)
    acc = jnp.zeros((R, 2 * H), jnp.float32)
    for t in range(T):
        gh = jnp.dot(h, whh, preferred_element_type=jnp.float32) + bhh
        gxt = gx[t]
        r = jax.nn.sigmoid(gxt[:, 0:128] + gh[:, 0:128])
        z = jax.nn.sigmoid(gxt[:, 128:256] + gh[:, 128:256])
        n = jnp.tanh(gxt[:, 256:384] + r * gh[:, 256:384])
        h = (1.0 - z) * n + z * h
        acc = acc + h
    out_ref[...] = acc * (1.0 / T)


def _gcn_kernel(a_ref, tm_ref, w1_ref, b1_ref, w2_ref, b2_ref,
                cw_ref, cb_ref, maskt_ref, out_ref):
    a = a_ref[...]
    w1 = w1_ref[...]
    w2 = w2_ref[...]
    b1 = b1_ref[...]
    b2 = b2_ref[...]
    cw = cw_ref[...]
    cb = cb_ref[0, 0]
    maskt = maskt_ref[...]
    for b in range(B):
        tm = tm_ref[b]
        u1 = jnp.dot(a, tm, preferred_element_type=jnp.float32)
        h1 = jnp.maximum(jnp.dot(u1, w1, preferred_element_type=jnp.float32) + b1, 0.0)
        u2 = jnp.dot(a, h1, preferred_element_type=jnp.float32)
        h2 = jnp.maximum(jnp.dot(u2, w2, preferred_element_type=jnp.float32) + b2, 0.0)
        h2t = jnp.transpose(h2)
        logits = jnp.dot(cw, h2t, preferred_element_type=jnp.float32) + cb
        masked = jnp.where(maskt == 0, -1e9, logits)
        m = jnp.max(masked, axis=1, keepdims=True)
        sh = masked - m
        lse = jnp.log(jnp.sum(jnp.exp(sh), axis=1, keepdims=True))
        out_ref[b] = sh - lse


def _blkdiag(a, b):
    z = jnp.zeros_like(a)
    return jnp.concatenate(
        [jnp.concatenate([a, z], axis=1), jnp.concatenate([z, b], axis=1)], axis=0)


@jax.jit
def kernel(x, edges, masks, W_ih_f, W_hh_f, b_ih_f, b_hh_f,
           W_ih_b, W_hh_b, b_ih_b, b_hh_b,
           gcn1_W, gcn1_b, gcn2_W, gcn2_b, cls_W, cls_b):
    wih_f, wih_b = W_ih_f.T, W_ih_b.T
    whh_f, whh_b = W_hh_f.T, W_hh_b.T
    wih = jnp.concatenate(
        [_blkdiag(wih_f[:, i * H:(i + 1) * H], wih_b[:, i * H:(i + 1) * H])
         for i in range(3)], axis=1)
    whh = jnp.concatenate(
        [_blkdiag(whh_f[:, i * H:(i + 1) * H], whh_b[:, i * H:(i + 1) * H])
         for i in range(3)], axis=1)
    bih = jnp.concatenate(
        [jnp.concatenate([b_ih_f[i * H:(i + 1) * H], b_ih_b[i * H:(i + 1) * H]])
         for i in range(3)]).reshape(1, 6 * H)
    bhh = jnp.concatenate(
        [jnp.concatenate([b_hh_f[i * H:(i + 1) * H], b_hh_b[i * H:(i + 1) * H]])
         for i in range(3)]).reshape(1, 6 * H)

    xct = x.reshape(B * N, T, F_IN).transpose(1, 0, 2)

    grid = (B * N) // R
    temporal = pl.pallas_call(
        _gru_kernel,
        grid=(grid,),
        in_specs=[
            pl.BlockSpec((T, R, F_IN), lambda i: (0, i, 0)),
            pl.BlockSpec((2 * F_IN, 6 * H), lambda i: (0, 0)),
            pl.BlockSpec((2 * H, 6 * H), lambda i: (0, 0)),
            pl.BlockSpec((1, 6 * H), lambda i: (0, 0)),
            pl.BlockSpec((1, 6 * H), lambda i: (0, 0)),
        ],
        out_specs=pl.BlockSpec((R, 2 * H), lambda i: (i, 0)),
        out_shape=jax.ShapeDtypeStruct((B * N, 2 * H), jnp.float32),
    )(xct, wih, whh, bih, bhh)

    tm = temporal.reshape(B, N, 2 * H)
    maskt = masks.T.astype(jnp.int32)

    preds = pl.pallas_call(
        _gcn_kernel,
        out_shape=jax.ShapeDtypeStruct((B, C, N), jnp.float32),
    )(edges, tm, gcn1_W, gcn1_b.reshape(1, 2 * H), gcn2_W,
      gcn2_b.reshape(1, 2 * H), cls_W.T, cls_b.reshape(1, 1), maskt)

    return preds


# speedup vs baseline: 3.1500x; 1.3026x over previous
import jax
import jax.numpy as jnp
from jax.experimental import pallas as pl

B = 2
N = 1000
T = 16
F_IN = 64
H = 64
C = 12
R = 400


def _gru_kernel(xct_ref, wih_ref, whh_ref, bih_ref, bhh_ref, out_ref):
    xbt = xct_ref[...]
    xrev = jnp.concatenate([xbt[T - 1 - t:T - t] for t in range(T)], axis=0)
    xc = jnp.concatenate([xbt, xrev], axis=-1)
    gx = jnp.dot(xc.reshape(T * R, 2 * F_IN), wih_ref[...],
                 preferred_element_type=jnp.float32) + bih_ref[...]
    gx = gx.reshape(T, R, 6 * H)
    whh = whh_ref[...]
    bhh = bhh_ref[...]
    h = jnp.zeros((R, 2 * H), jnp.float32)
    acc = jnp.zeros((R, 2 * H), jnp.float32)
    for t in range(T):
        gh = jnp.dot(h.astype(jnp.bfloat16), whh,
                     preferred_element_type=jnp.float32) + bhh
        gxt = gx[t]
        r = jax.nn.sigmoid(gxt[:, 0:128] + gh[:, 0:128])
        z = jax.nn.sigmoid(gxt[:, 128:256] + gh[:, 128:256])
        n = jnp.tanh(gxt[:, 256:384] + r * gh[:, 256:384])
        h = (1.0 - z) * n + z * h
        acc = acc + h
    out_ref[...] = acc * (1.0 / T)


def _gcn_kernel(a_ref, tm_ref, w1_ref, b1_ref, w2_ref, b2_ref,
                cw_ref, cb_ref, maskt_ref, out_ref):
    a = a_ref[...]
    w1 = w1_ref[...]
    w2 = w2_ref[...]
    b1 = b1_ref[...]
    b2 = b2_ref[...]
    cw = cw_ref[...]
    cb = cb_ref[0, 0]
    maskt = maskt_ref[...]
    for b in range(B):
        tm = tm_ref[b]
        u1 = jnp.dot(a, tm, preferred_element_type=jnp.float32)
        h1 = jnp.maximum(jnp.dot(u1, w1, preferred_element_type=jnp.float32) + b1, 0.0)
        u2 = jnp.dot(a, h1, preferred_element_type=jnp.float32)
        h2 = jnp.maximum(jnp.dot(u2, w2, preferred_element_type=jnp.float32) + b2, 0.0)
        h2t = jnp.transpose(h2)
        logits = jnp.dot(cw, h2t, preferred_element_type=jnp.float32) + cb
        masked = jnp.where(maskt == 0, -1e9, logits)
        m = jnp.max(masked, axis=1, keepdims=True)
        sh = masked - m
        lse = jnp.log(jnp.sum(jnp.exp(sh), axis=1, keepdims=True))
        out_ref[b] = sh - lse


def _blkdiag(a, b):
    z = jnp.zeros_like(a)
    return jnp.concatenate(
        [jnp.concatenate([a, z], axis=1), jnp.concatenate([z, b], axis=1)], axis=0)


@jax.jit
def kernel(x, edges, masks, W_ih_f, W_hh_f, b_ih_f, b_hh_f,
           W_ih_b, W_hh_b, b_ih_b, b_hh_b,
           gcn1_W, gcn1_b, gcn2_W, gcn2_b, cls_W, cls_b):
    wih_f, wih_b = W_ih_f.T, W_ih_b.T
    whh_f, whh_b = W_hh_f.T, W_hh_b.T
    wih = jnp.concatenate(
        [_blkdiag(wih_f[:, i * H:(i + 1) * H], wih_b[:, i * H:(i + 1) * H])
         for i in range(3)], axis=1)
    whh = jnp.concatenate(
        [_blkdiag(whh_f[:, i * H:(i + 1) * H], whh_b[:, i * H:(i + 1) * H])
         for i in range(3)], axis=1)
    bih = jnp.concatenate(
        [jnp.concatenate([b_ih_f[i * H:(i + 1) * H], b_ih_b[i * H:(i + 1) * H]])
         for i in range(3)]).reshape(1, 6 * H)
    bhh = jnp.concatenate(
        [jnp.concatenate([b_hh_f[i * H:(i + 1) * H], b_hh_b[i * H:(i + 1) * H]])
         for i in range(3)]).reshape(1, 6 * H)

    xct = x.astype(jnp.bfloat16).reshape(B * N, T, F_IN).transpose(1, 0, 2)

    grid = (B * N) // R
    temporal = pl.pallas_call(
        _gru_kernel,
        grid=(grid,),
        in_specs=[
            pl.BlockSpec((T, R, F_IN), lambda i: (0, i, 0)),
            pl.BlockSpec((2 * F_IN, 6 * H), lambda i: (0, 0)),
            pl.BlockSpec((2 * H, 6 * H), lambda i: (0, 0)),
            pl.BlockSpec((1, 6 * H), lambda i: (0, 0)),
            pl.BlockSpec((1, 6 * H), lambda i: (0, 0)),
        ],
        out_specs=pl.BlockSpec((R, 2 * H), lambda i: (i, 0)),
        out_shape=jax.ShapeDtypeStruct((B * N, 2 * H), jnp.float32),
    )(xct, wih.astype(jnp.bfloat16), whh.astype(jnp.bfloat16), bih, bhh)

    tm = temporal.reshape(B, N, 2 * H)
    maskt = masks.T.astype(jnp.int32)

    preds = pl.pallas_call(
        _gcn_kernel,
        out_shape=jax.ShapeDtypeStruct((B, C, N), jnp.float32),
    )(edges, tm, gcn1_W, gcn1_b.reshape(1, 2 * H), gcn2_W,
      gcn2_b.reshape(1, 2 * H), cls_W.T, cls_b.reshape(1, 1), maskt)

    return preds
